# Initial kernel scaffold; baseline (speedup 1.0000x reference)
import jax
import jax.numpy as jnp
from jax.experimental import pallas as pl


def kernel(crime_embedding, row, col, vals, rw_row, rw_col, rw_vals, s_W1, s_b1, s_W2, s_b2, t_W1, t_b1, t_W2, t_b2, t_eps, s_eps):
    raise NotImplementedError("write your pallas kernel here")


# speedup vs baseline: 1.9319x; 1.9319x over previous
import functools

import jax
import jax.numpy as jnp
from jax import lax
from jax.experimental import pallas as pl
from jax.experimental.pallas import tpu as pltpu
from jax.experimental.pallas import tpu_sc as plsc

L = 2
N = 10000
D = 128
DH = D // 2
WALK = 8
TMP = 0.5

NC = 2
NS = 16
CHUNK = 128
STRIPE = 624


@functools.lru_cache(maxsize=None)
def _make_spmm(n_chunks: int):
    mesh = plsc.VectorSubcoreMesh(
        core_axis_name="c", subcore_axis_name="s", num_cores=NC, num_subcores=NS
    )

    @functools.partial(
        pl.kernel,
        out_type=jax.ShapeDtypeStruct((NC, N, DH), jnp.float32),
        mesh=mesh,
        scratch_types=[
            pltpu.VMEM((n_chunks, CHUNK), jnp.int32),
            pltpu.VMEM((n_chunks, CHUNK), jnp.int32),
            pltpu.VMEM((n_chunks, CHUNK), jnp.float32),
            pltpu.VMEM((CHUNK, DH), jnp.float32),
            pltpu.VMEM_SHARED((N, DH), jnp.float32),
            pltpu.SemaphoreType.DMA,
        ],
        compiler_params=pltpu.CompilerParams(use_tc_tiling_on_sc=False),
    )
    def spmm(x_hbm, col_hbm, row_hbm, w_hbm, out_hbm, colv, rowv, wv, gbuf, acc, sem):
        c = lax.axis_index("c")
        s = lax.axis_index("s")

        pltpu.sync_copy(col_hbm.at[s], colv)
        pltpu.sync_copy(row_hbm.at[s], rowv)
        pltpu.sync_copy(w_hbm.at[s], wv)

        def fix_body(k, _):
            for j in range(CHUNK // 16):
                sl = (k, pl.ds(j * 16, 16))
                colv[sl] = colv[sl] + c * N
            return 0

        lax.fori_loop(0, n_chunks, fix_body, 0, unroll=False)

        zero16 = jnp.zeros((16,), jnp.float32)

        def zero_body(e, _):
            for j in range(DH // 16):
                gbuf[e, pl.ds(j * 16, 16)] = zero16
            return 0

        lax.fori_loop(0, CHUNK, zero_body, 0, unroll=False)
        for t in range(5):
            pltpu.sync_copy(
                gbuf.at[pl.ds(0, CHUNK)],
                acc.at[pl.ds(s * STRIPE + t * CHUNK, CHUNK)],
            )
        plsc.subcore_barrier()

        def chunk_body(k, _):
            pltpu.async_copy(x_hbm.at[colv.at[k]], gbuf, sem).wait()

            def scale_body(g, _):
                w16 = wv[k, pl.ds(g * 16, 16)]
                for e16 in range(16):
                    w = w16[e16]
                    e = g * 16 + e16
                    for j in range(DH // 16):
                        sl = (e, pl.ds(j * 16, 16))
                        gbuf[sl] = gbuf[sl] * w
                return 0

            lax.fori_loop(0, CHUNK // 16, scale_body, 0, unroll=False)
            pltpu.sync_copy(gbuf, acc.at[rowv.at[k]], add=True)
            return 0

        lax.fori_loop(0, n_chunks, chunk_body, 0, unroll=False)
        plsc.subcore_barrier()

        pltpu.sync_copy(
            acc.at[pl.ds(s * STRIPE, STRIPE)],
            out_hbm.at[c, pl.ds(s * STRIPE, STRIPE)],
        )

        @pl.when(s == NS - 1)
        def _tail():
            pltpu.sync_copy(
                acc.at[pl.ds(NS * STRIPE, N - NS * STRIPE)],
                out_hbm.at[c, pl.ds(NS * STRIPE, N - NS * STRIPE)],
            )

    return spmm


def _pad_edges(a, n_chunks):
    npad = NS * n_chunks * CHUNK - a.shape[0]
    return jnp.pad(a, (0, npad)).reshape(NS, n_chunks, CHUNK)


def _n_chunks(nnz):
    return -(-nnz // (NS * CHUNK))


def kernel(crime_embedding, row, col, vals, rw_row, rw_col, rw_vals,
           s_W1, s_b1, s_W2, s_b2, t_W1, t_b1, t_W2, t_b2, t_eps, s_eps):
    X0 = crime_embedding
    nnz = row.shape[0]
    nch = _n_chunks(nnz)
    nch_rw = _n_chunks(rw_row.shape[0])
    spmm_main = _make_spmm(nch)
    spmm_rw = _make_spmm(nch_rw)

    rowp = _pad_edges(row, nch)
    colp = _pad_edges(col, nch)
    rw_rowp = _pad_edges(rw_row, nch_rw)
    rw_colp = _pad_edges(rw_col, nch_rw)
    rw_valsp = _pad_edges(rw_vals, nch_rw)

    def split(x):
        return x.reshape(N, 2, DH).transpose(1, 0, 2).reshape(2 * N, DH)

    def unsplit(o):
        return jnp.concatenate([o[0], o[1]], axis=-1)

    def spmm(x_split, w):
        o = spmm_main(x_split, colp, rowp, _pad_edges(w, nch))
        return o.reshape(2 * N, DH), unsplit(o)

    def rw_mean(x_split):
        o = spmm_rw(x_split, rw_colp, rw_rowp, rw_valsp)
        return o.reshape(2 * N, DH), unsplit(o)

    X0s = split(X0)
    E1s, E1 = spmm(X0s, vals)
    E2s, E2 = spmm(E1s, vals)

    t_masks = []
    for i in range(L):
        A = X0 @ t_W1[i][:D]
        B = X0 @ t_W1[i][D:] + t_b1[i]
        h = jax.nn.relu(A[row] + B[col])
        tm = h @ t_W2[i] + t_b2[i]
        eps = t_eps[i]
        gate = (jnp.log(eps) - jnp.log(1 - eps) + tm) / TMP
        t_masks.append(jax.nn.sigmoid(gate).squeeze(1))

    s_masks = []
    for i, E in enumerate((E1, E2)):
        h2 = jax.nn.relu(E @ s_W1[i] + s_b1[i])
        sm = h2 @ s_W2[i] + s_b2[i]
        eps2 = s_eps[i]
        gate = (jnp.log(eps2) - jnp.log(1 - eps2) + sm) / TMP
        s_masks.append(jax.nn.sigmoid(gate))

    T1s, T1 = spmm(X0s, vals * t_masks[0])
    _, T2 = spmm(T1s, vals * t_masks[1])
    out_t = (X0 + T1 + T2) / 3.0

    Ss, S = X0s, X0
    acc_s = X0
    s_reg = jnp.zeros((), jnp.float32)
    for i in range(L):
        _, mp = rw_mean(Ss)
        Cmb = s_masks[i] * S + (1 - s_masks[i]) * mp
        Ss, S = spmm(split(Cmb), vals)
        acc_s = acc_s + S
        s_reg = s_reg + s_masks[i].sum() / N
    out_s = acc_s / 3.0
    s_reg = s_reg / L

    t_reg = jnp.zeros((), jnp.float32)
    return (out_t, out_s, t_reg, s_reg, t_masks[-1])


# speedup vs baseline: 2.4200x; 1.2527x over previous
import functools

import jax
import jax.numpy as jnp
from jax import lax
from jax.experimental import pallas as pl
from jax.experimental.pallas import tpu as pltpu
from jax.experimental.pallas import tpu_sc as plsc

L = 2
N = 10000
D = 128
DH = D // 2
TMP = 0.5

NC = 2
NS = 16
CHUNK = 128
STRIPE = 624
BLKR = 400

_SC_PARAMS = pltpu.CompilerParams(use_tc_tiling_on_sc=False)
_SC_PARAMS_NOLAYOUT = pltpu.CompilerParams(
    use_tc_tiling_on_sc=False, needs_layout_passes=False
)


def _sc_mesh():
    return plsc.VectorSubcoreMesh(
        core_axis_name="c", subcore_axis_name="s", num_cores=NC, num_subcores=NS
    )


@functools.lru_cache(maxsize=None)
def _make_spmm(n_chunks: int):
    assert n_chunks % 2 == 0

    @functools.partial(
        pl.kernel,
        out_type=jax.ShapeDtypeStruct((NC, N, DH), jnp.float32),
        mesh=_sc_mesh(),
        scratch_types=[
            pltpu.VMEM((n_chunks, CHUNK), jnp.int32),
            pltpu.VMEM((n_chunks, CHUNK), jnp.float32),
            pltpu.VMEM((2, CHUNK), jnp.int32),
            pltpu.VMEM((2, CHUNK), jnp.int32),
            pltpu.VMEM((CHUNK, DH), jnp.float32),
            pltpu.VMEM((CHUNK, DH), jnp.float32),
            pltpu.VMEM((CHUNK, DH), jnp.float32),
            pltpu.VMEM((CHUNK, DH), jnp.float32),
            pltpu.VMEM_SHARED((N, DH), jnp.float32),
            pltpu.SemaphoreType.DMA((2,)),
            pltpu.SemaphoreType.DMA((2,)),
        ],
        compiler_params=_SC_PARAMS,
    )
    def spmm(x_hbm, pack_hbm, w_hbm, out_hbm,
             packv, wv, colr, rowr, gbuf0, gbuf1, sbuf0, sbuf1, acc,
             sem_g, sem_s):
        gbufs = (gbuf0, gbuf1)
        sbufs = (sbuf0, sbuf1)
        c = lax.axis_index("c")
        s = lax.axis_index("s")

        pltpu.sync_copy(pack_hbm.at[s], packv)
        pltpu.sync_copy(w_hbm.at[s], wv)

        def unpack_col(k, b):
            for j in range(CHUNK // 16):
                p = packv[k, pl.ds(j * 16, 16)]
                colr[b, pl.ds(j * 16, 16)] = (p & 0xFFFF) + c * N

        def unpack_row(k, b):
            for j in range(CHUNK // 16):
                p = packv[k, pl.ds(j * 16, 16)]
                rowr[b, pl.ds(j * 16, 16)] = p >> 16

        zero16 = jnp.zeros((16,), jnp.float32)

        def zero_body(e, _):
            for j in range(DH // 16):
                gbuf0[e, pl.ds(j * 16, 16)] = zero16
            return 0

        lax.fori_loop(0, CHUNK, zero_body, 0, unroll=False)
        for t in range(5):
            pltpu.sync_copy(
                gbuf0.at[...], acc.at[pl.ds(s * STRIPE + t * CHUNK, CHUNK)]
            )
        plsc.subcore_barrier()

        for b in range(2):
            unpack_col(b, b)
            pltpu.async_copy(x_hbm.at[colr.at[b]], gbufs[b], sem_g.at[b])

        def pair_body(kk, _):
            for b in range(2):
                k = kk * 2 + b
                pltpu.make_async_copy(
                    x_hbm.at[colr.at[b]], gbufs[b], sem_g.at[b]
                ).wait()

                @pl.when(k >= 2)
                def _wait_prev_scatter():
                    pltpu.make_async_copy(
                        sbufs[b], acc.at[rowr.at[b]], sem_s.at[b]
                    ).wait()

                unpack_row(k, b)

                def scale_body(g, _):
                    w16 = wv[k, pl.ds(g * 16, 16)]
                    for e16 in range(16):
                        w = w16[e16]
                        e = g * 16 + e16
                        for j in range(DH // 16):
                            sbufs[b][e, pl.ds(j * 16, 16)] = (
                                gbufs[b][e, pl.ds(j * 16, 16)] * w
                            )
                    return 0

                lax.fori_loop(0, CHUNK // 16, scale_body, 0, unroll=False)

                @pl.when(k + 2 < n_chunks)
                def _issue_next_gather():
                    unpack_col(k + 2, b)
                    pltpu.async_copy(
                        x_hbm.at[colr.at[b]], gbufs[b], sem_g.at[b]
                    )

                pltpu.async_copy(
                    sbufs[b], acc.at[rowr.at[b]], sem_s.at[b], add=True
                )
            return 0

        lax.fori_loop(0, n_chunks // 2, pair_body, 0, unroll=False)
        for b in range(2):
            pltpu.make_async_copy(
                sbufs[b], acc.at[rowr.at[b]], sem_s.at[b]
            ).wait()
        plsc.subcore_barrier()

        pltpu.sync_copy(
            acc.at[pl.ds(s * STRIPE, STRIPE)],
            out_hbm.at[c, pl.ds(s * STRIPE, STRIPE)],
        )

        @pl.when(s == NS - 1)
        def _tail():
            pltpu.sync_copy(
                acc.at[pl.ds(NS * STRIPE, N - NS * STRIPE)],
                out_hbm.at[c, pl.ds(NS * STRIPE, N - NS * STRIPE)],
            )

    return spmm


@functools.lru_cache(maxsize=None)
def _make_edge_logit(n_chunks: int):
    assert n_chunks % 2 == 0

    @functools.partial(
        pl.kernel,
        out_type=jax.ShapeDtypeStruct((NC, NS, n_chunks, CHUNK), jnp.float32),
        mesh=_sc_mesh(),
        scratch_types=[
            pltpu.VMEM((n_chunks, CHUNK), jnp.int32),
            pltpu.VMEM((n_chunks, CHUNK), jnp.int32),
            pltpu.VMEM((CHUNK, DH), jnp.float32),
            pltpu.VMEM((CHUNK, DH), jnp.float32),
            pltpu.VMEM((CHUNK, DH), jnp.float32),
            pltpu.VMEM((CHUNK, DH), jnp.float32),
            pltpu.VMEM((n_chunks, CHUNK), jnp.float32),
            pltpu.VMEM((DH,), jnp.float32),
            pltpu.SemaphoreType.DMA((2,)),
            pltpu.SemaphoreType.DMA((2,)),
        ],
        compiler_params=_SC_PARAMS_NOLAYOUT,
    )
    def elog(ab_hbm, aidx_hbm, bidx_hbm, w2_hbm, out_hbm,
             arow, bcol, abuf0, abuf1, bbuf0, bbuf1, obuf, w2v, sem_a, sem_b):
        abufs = (abuf0, abuf1)
        bbufs = (bbuf0, bbuf1)
        c = lax.axis_index("c")
        s = lax.axis_index("s")

        pltpu.sync_copy(aidx_hbm.at[c, s], arow)
        pltpu.sync_copy(bidx_hbm.at[c, s], bcol)
        pltpu.sync_copy(w2_hbm.at[pl.ds(c * DH, DH)], w2v)

        for b in range(2):
            pltpu.async_copy(ab_hbm.at[arow.at[b]], abufs[b], sem_a.at[b])
            pltpu.async_copy(ab_hbm.at[bcol.at[b]], bbufs[b], sem_b.at[b])

        lane = lax.iota(jnp.int32, 16)

        def pair_body(kk, _):
            for b in range(2):
                k = kk * 2 + b
                pltpu.make_async_copy(
                    ab_hbm.at[arow.at[k]], abufs[b], sem_a.at[b]
                ).wait()
                pltpu.make_async_copy(
                    ab_hbm.at[bcol.at[k]], bbufs[b], sem_b.at[b]
                ).wait()

                def group_body(g, _):
                    erow = g * 16 + lane
                    acc = jnp.zeros((16,), jnp.float32)
                    for j in range(DH // 16):
                        w16 = w2v[pl.ds(j * 16, 16)]
                        for l in range(16):
                            kf = jnp.full((16,), j * 16 + l, jnp.int32)
                            av = plsc.load_gather(abufs[b], [erow, kf])
                            bv = plsc.load_gather(bbufs[b], [erow, kf])
                            acc = acc + jnp.maximum(av + bv, 0.0) * w16[l]
                    obuf[k, pl.ds(g * 16, 16)] = acc
                    return 0

                lax.fori_loop(0, CHUNK // 16, group_body, 0, unroll=False)

                @pl.when(k + 2 < n_chunks)
                def _issue_next():
                    pltpu.async_copy(
                        ab_hbm.at[arow.at[k + 2]], abufs[b], sem_a.at[b]
                    )
                    pltpu.async_copy(
                        ab_hbm.at[bcol.at[k + 2]], bbufs[b], sem_b.at[b]
                    )
            return 0

        lax.fori_loop(0, n_chunks // 2, pair_body, 0, unroll=False)
        pltpu.sync_copy(obuf, out_hbm.at[c, s])

    return elog


def _tc_prep(X0, W4, b4):
    def body(x_ref, w_ref, b_ref, out_ref):
        out_ref[0, 0] = (
            jnp.dot(x_ref[...], w_ref[0, 0], preferred_element_type=jnp.float32)
            + b_ref[0, 0, 0]
        )

    return pl.pallas_call(
        body,
        grid=(4, NC, N // BLKR),
        in_specs=[
            pl.BlockSpec((BLKR, D), lambda m, ci, r: (r, 0)),
            pl.BlockSpec((1, 1, D, DH), lambda m, ci, r: (m, ci, 0, 0)),
            pl.BlockSpec((1, 1, 1, DH), lambda m, ci, r: (m, ci, 0, 0)),
        ],
        out_specs=pl.BlockSpec((1, 1, BLKR, DH), lambda m, ci, r: (m, ci, r, 0)),
        out_shape=jax.ShapeDtypeStruct((4, NC, N, DH), jnp.float32),
    )(X0, W4, b4)


def _tc_smask(E_split, w1, b1, w2, b2, eps):
    def body(e_ref, w1_ref, b1_ref, w2_ref, b2_ref, eps_ref, sm_ref, ps_ref):
        E = jnp.concatenate([e_ref[0], e_ref[1]], axis=-1)
        h = jnp.maximum(
            jnp.dot(E, w1_ref[...], preferred_element_type=jnp.float32)
            + b1_ref[...],
            0.0,
        )
        sm = jnp.dot(h, w2_ref[...], preferred_element_type=jnp.float32) + b2_ref[0]
        e = eps_ref[...]
        gate = (jnp.log(e) - jnp.log(1.0 - e) + sm) / TMP
        m = jax.nn.sigmoid(gate)
        sm_ref[...] = m
        ps_ref[pl.program_id(0), 0] = jnp.sum(m)

    return pl.pallas_call(
        body,
        grid=(N // BLKR,),
        in_specs=[
            pl.BlockSpec((NC, BLKR, DH), lambda r: (0, r, 0)),
            pl.BlockSpec((D, D), lambda r: (0, 0)),
            pl.BlockSpec((1, D), lambda r: (0, 0)),
            pl.BlockSpec((D, 1), lambda r: (0, 0)),
            pl.BlockSpec(memory_space=pltpu.SMEM),
            pl.BlockSpec((BLKR, 1), lambda r: (r, 0)),
        ],
        out_specs=[
            pl.BlockSpec((BLKR, 1), lambda r: (r, 0)),
            pl.BlockSpec(memory_space=pltpu.SMEM),
        ],
        out_shape=[
            jax.ShapeDtypeStruct((N, 1), jnp.float32),
            jax.ShapeDtypeStruct((N // BLKR, 1), jnp.float32),
        ],
    )(E_split, w1, b1, w2, b2, eps)


def _tc_tgate(parts, eps2, vals2, b2):
    M = eps2.shape[0]

    def body(p_ref, eps_ref, v_ref, b2_ref, tm_ref, wv_ref):
        tm = p_ref[0] + p_ref[1] + b2_ref[0]
        e = eps_ref[...]
        gate = (jnp.log(e) - jnp.log(1.0 - e) + tm) / TMP
        m = jax.nn.sigmoid(gate)
        tm_ref[...] = m
        wv_ref[...] = v_ref[...] * m

    return pl.pallas_call(
        body,
        in_specs=[
            pl.BlockSpec((NC, M, CHUNK), lambda: (0, 0, 0)),
            pl.BlockSpec((M, CHUNK), lambda: (0, 0)),
            pl.BlockSpec((M, CHUNK), lambda: (0, 0)),
            pl.BlockSpec(memory_space=pltpu.SMEM),
        ],
        out_specs=[
            pl.BlockSpec((M, CHUNK), lambda: (0, 0)),
            pl.BlockSpec((M, CHUNK), lambda: (0, 0)),
        ],
        out_shape=[
            jax.ShapeDtypeStruct((M, CHUNK), jnp.float32),
            jax.ShapeDtypeStruct((M, CHUNK), jnp.float32),
        ],
    )(parts, eps2, vals2, b2)


def _tc_combine(S_split, mp_split, sm):
    def body(s_ref, mp_ref, sm_ref, out_ref):
        smv = sm_ref[...]
        out_ref[0] = smv * s_ref[0] + (1.0 - smv) * mp_ref[0]

    return pl.pallas_call(
        body,
        grid=(NC, N // BLKR),
        in_specs=[
            pl.BlockSpec((1, BLKR, DH), lambda ci, r: (ci, r, 0)),
            pl.BlockSpec((1, BLKR, DH), lambda ci, r: (ci, r, 0)),
            pl.BlockSpec((BLKR, 1), lambda ci, r: (r, 0)),
        ],
        out_specs=pl.BlockSpec((1, BLKR, DH), lambda ci, r: (ci, r, 0)),
        out_shape=jax.ShapeDtypeStruct((NC, N, DH), jnp.float32),
    )(S_split, mp_split, sm)


def _tc_final(X0, T1, T2, S1, S2):
    def body(x_ref, t1_ref, t2_ref, s1_ref, s2_ref, ot_ref, os_ref):
        def merge(r):
            return jnp.concatenate([r[0], r[1]], axis=-1)

        x = x_ref[...]
        ot_ref[...] = (x + merge(t1_ref) + merge(t2_ref)) * (1.0 / 3.0)
        os_ref[...] = (x + merge(s1_ref) + merge(s2_ref)) * (1.0 / 3.0)

    split_spec = pl.BlockSpec((NC, BLKR, DH), lambda r: (0, r, 0))
    nat_spec = pl.BlockSpec((BLKR, D), lambda r: (r, 0))
    return pl.pallas_call(
        body,
        grid=(N // BLKR,),
        in_specs=[nat_spec, split_spec, split_spec, split_spec, split_spec],
        out_specs=[nat_spec, nat_spec],
        out_shape=[
            jax.ShapeDtypeStruct((N, D), jnp.float32),
            jax.ShapeDtypeStruct((N, D), jnp.float32),
        ],
    )(X0, T1, T2, S1, S2)


def _n_chunks(nnz):
    nch = -(-nnz // (NS * CHUNK))
    return nch + (nch % 2)


def _pad_edges(a, n_chunks):
    npad = NS * n_chunks * CHUNK - a.shape[0]
    return jnp.pad(a, (0, npad)).reshape(NS, n_chunks, CHUNK)


def kernel(crime_embedding, row, col, vals, rw_row, rw_col, rw_vals,
           s_W1, s_b1, s_W2, s_b2, t_W1, t_b1, t_W2, t_b2, t_eps, s_eps):
    X0 = crime_embedding
    nnz = row.shape[0]
    nch = _n_chunks(nnz)
    nch_rw = _n_chunks(rw_row.shape[0])
    spmm_main = _make_spmm(nch)
    spmm_rw = _make_spmm(nch_rw)

    rowp = _pad_edges(row, nch)
    colp = _pad_edges(col, nch)
    packp = _pad_edges((row << 16) | col, nch)
    rw_packp = _pad_edges((rw_row << 16) | rw_col, nch_rw)
    rw_valsp = _pad_edges(rw_vals, nch_rw)

    def spmm(x_split, w):
        o = spmm_main(x_split.reshape(2 * N, DH), packp, _pad_edges(w, nch))
        return o

    X0s = X0.reshape(N, 2, DH).transpose(1, 0, 2)

    W4 = jnp.stack([
        jnp.stack([t_W1[i][half * D:(half + 1) * D, ci * DH:(ci + 1) * DH]
                   for ci in range(NC)])
        for i in range(L) for half in range(2)
    ])
    b4 = jnp.stack([
        jnp.stack([jnp.where(half == 1, t_b1[i][ci * DH:(ci + 1) * DH], 0.0)
                   for ci in range(NC)])
        for i in range(L) for half in range(2)
    ])
    AB = _tc_prep(X0, W4, b4.reshape(4, NC, 1, DH))

    M = nnz // CHUNK
    t_masks, wvals = [], []
    for i in range(L):
        aidx = rowp[None] + jnp.array([4 * i * N, (4 * i + 1) * N],
                                      jnp.int32).reshape(NC, 1, 1, 1)
        bidx = colp[None] + jnp.array([(4 * i + 2) * N, (4 * i + 3) * N],
                                      jnp.int32).reshape(NC, 1, 1, 1)
        parts = _make_edge_logit(nch)(
            AB.reshape(4 * NC * N, DH), aidx, bidx, t_W2[i][:, 0]
        )
        parts = parts.reshape(NC, NS * nch * CHUNK)[:, :nnz].reshape(NC, M, CHUNK)
        tm, wv = _tc_tgate(
            parts,
            t_eps[i].reshape(M, CHUNK),
            vals.reshape(M, CHUNK),
            t_b2[i].reshape(1),
        )
        t_masks.append(tm.reshape(nnz))
        wvals.append(wv.reshape(nnz))

    E1 = spmm(X0s, vals)
    E2 = spmm(E1, vals)
    s_masks, s_partials = [], []
    for i, E in enumerate((E1, E2)):
        sm, ps = _tc_smask(
            E, s_W1[i], s_b1[i].reshape(1, D), s_W2[i], s_b2[i], s_eps[i]
        )
        s_masks.append(sm)
        s_partials.append(ps)

    T1 = spmm(X0s, wvals[0])
    T2 = spmm(T1, wvals[1])

    S = X0s
    outs = []
    for i in range(L):
        mp = spmm_rw(S.reshape(2 * N, DH), rw_packp, rw_valsp)
        C = _tc_combine(S, mp, s_masks[i])
        S = spmm(C, vals)
        outs.append(S)

    out_t, out_s = _tc_final(X0, T1, T2, outs[0], outs[1])

    s_reg = (s_partials[0].sum() + s_partials[1].sum()) / N / L
    t_reg = jnp.zeros((), jnp.float32)
    return (out_t, out_s, t_reg, s_reg, t_masks[-1])


# speedup vs baseline: 4.0666x; 1.6804x over previous
import functools

import jax
import jax.numpy as jnp
from jax import lax
from jax.experimental import pallas as pl
from jax.experimental.pallas import tpu as pltpu
from jax.experimental.pallas import tpu_sc as plsc

L = 2
N = 10000
D = 128
DH = D // 2
TMP = 0.5

NC = 2
NS = 16
CHUNK = 128
STRIPE = 624
BLKR = 400

_SC_PARAMS = pltpu.CompilerParams(use_tc_tiling_on_sc=False)
_SC_PARAMS_NOLAYOUT = pltpu.CompilerParams(
    use_tc_tiling_on_sc=False, needs_layout_passes=False
)


def _sc_mesh():
    return plsc.VectorSubcoreMesh(
        core_axis_name="c", subcore_axis_name="s", num_cores=NC, num_subcores=NS
    )


@functools.lru_cache(maxsize=None)
def _make_spmm(n_chunks: int):
    assert n_chunks % 2 == 0

    @functools.partial(
        pl.kernel,
        out_type=jax.ShapeDtypeStruct((NC, N, DH), jnp.float32),
        mesh=_sc_mesh(),
        scratch_types=[
            pltpu.VMEM((n_chunks, CHUNK), jnp.int32),
            pltpu.VMEM((n_chunks, CHUNK), jnp.float32),
            pltpu.VMEM((2, CHUNK), jnp.int32),
            pltpu.VMEM((2, CHUNK), jnp.int32),
            pltpu.VMEM((CHUNK, DH), jnp.float32),
            pltpu.VMEM((CHUNK, DH), jnp.float32),
            pltpu.VMEM((CHUNK, DH), jnp.float32),
            pltpu.VMEM((CHUNK, DH), jnp.float32),
            pltpu.VMEM_SHARED((N, DH), jnp.float32),
            pltpu.SemaphoreType.DMA((2,)),
            pltpu.SemaphoreType.DMA((2,)),
        ],
        compiler_params=_SC_PARAMS,
    )
    def spmm(x_hbm, pack_hbm, w_hbm, out_hbm,
             packv, wv, colr, rowr, gbuf0, gbuf1, sbuf0, sbuf1, acc,
             sem_g, sem_s):
        gbufs = (gbuf0, gbuf1)
        sbufs = (sbuf0, sbuf1)
        c = lax.axis_index("c")
        s = lax.axis_index("s")

        pltpu.sync_copy(pack_hbm.at[s], packv)
        pltpu.sync_copy(w_hbm.at[s], wv)

        def unpack_col(k, b):
            for j in range(CHUNK // 16):
                p = packv[k, pl.ds(j * 16, 16)]
                colr[b, pl.ds(j * 16, 16)] = (p & 0xFFFF) + c * N

        def unpack_row(k, b):
            for j in range(CHUNK // 16):
                p = packv[k, pl.ds(j * 16, 16)]
                rowr[b, pl.ds(j * 16, 16)] = p >> 16

        zero16 = jnp.zeros((16,), jnp.float32)

        def zero_body(e, _):
            for j in range(DH // 16):
                gbuf0[e, pl.ds(j * 16, 16)] = zero16
            return 0

        lax.fori_loop(0, CHUNK, zero_body, 0, unroll=False)
        for t in range(5):
            pltpu.sync_copy(
                gbuf0.at[...], acc.at[pl.ds(s * STRIPE + t * CHUNK, CHUNK)]
            )
        plsc.subcore_barrier()

        for b in range(2):
            unpack_col(b, b)
            pltpu.async_copy(x_hbm.at[colr.at[b]], gbufs[b], sem_g.at[b])

        def pair_body(kk, _):
            for b in range(2):
                k = kk * 2 + b
                pltpu.make_async_copy(
                    x_hbm.at[colr.at[b]], gbufs[b], sem_g.at[b]
                ).wait()

                @pl.when(k >= 2)
                def _wait_prev_scatter():
                    pltpu.make_async_copy(
                        sbufs[b], acc.at[rowr.at[b]], sem_s.at[b]
                    ).wait()

                unpack_row(k, b)

                def scale_body(g, _):
                    w16 = wv[k, pl.ds(g * 16, 16)]
                    for e16 in range(16):
                        w = w16[e16]
                        e = g * 16 + e16
                        for j in range(DH // 16):
                            sbufs[b][e, pl.ds(j * 16, 16)] = (
                                gbufs[b][e, pl.ds(j * 16, 16)] * w
                            )
                    return 0

                lax.fori_loop(0, CHUNK // 16, scale_body, 0, unroll=False)

                @pl.when(k + 2 < n_chunks)
                def _issue_next_gather():
                    unpack_col(k + 2, b)
                    pltpu.async_copy(
                        x_hbm.at[colr.at[b]], gbufs[b], sem_g.at[b]
                    )

                pltpu.async_copy(
                    sbufs[b], acc.at[rowr.at[b]], sem_s.at[b], add=True
                )
            return 0

        lax.fori_loop(0, n_chunks // 2, pair_body, 0, unroll=False)
        for b in range(2):
            pltpu.make_async_copy(
                sbufs[b], acc.at[rowr.at[b]], sem_s.at[b]
            ).wait()
        plsc.subcore_barrier()

        pltpu.sync_copy(
            acc.at[pl.ds(s * STRIPE, STRIPE)],
            out_hbm.at[c, pl.ds(s * STRIPE, STRIPE)],
        )

        @pl.when(s == NS - 1)
        def _tail():
            pltpu.sync_copy(
                acc.at[pl.ds(NS * STRIPE, N - NS * STRIPE)],
                out_hbm.at[c, pl.ds(NS * STRIPE, N - NS * STRIPE)],
            )

    return spmm


@functools.lru_cache(maxsize=None)
def _make_edge_logit(n_chunks: int):
    assert n_chunks % 2 == 0

    @functools.partial(
        pl.kernel,
        out_type=jax.ShapeDtypeStruct((NC, NS, n_chunks, CHUNK), jnp.float32),
        mesh=_sc_mesh(),
        scratch_types=[
            pltpu.VMEM((n_chunks, CHUNK), jnp.int32),
            pltpu.VMEM((n_chunks, CHUNK), jnp.int32),
            pltpu.VMEM((CHUNK, DH), jnp.float32),
            pltpu.VMEM((CHUNK, DH), jnp.float32),
            pltpu.VMEM((CHUNK, DH), jnp.float32),
            pltpu.VMEM((CHUNK, DH), jnp.float32),
            pltpu.VMEM((n_chunks, CHUNK), jnp.float32),
            pltpu.VMEM((DH * 16,), jnp.float32),
            pltpu.VMEM((DH * 16,), jnp.int32),
            pltpu.SemaphoreType.DMA((2,)),
            pltpu.SemaphoreType.DMA((2,)),
        ],
        compiler_params=_SC_PARAMS_NOLAYOUT,
    )
    def elog(ab_hbm, aidx_hbm, bidx_hbm, w2rot_hbm, featrot_hbm, out_hbm,
             arow, bcol, abuf0, abuf1, bbuf0, bbuf1, obuf, w2v, featv_tab,
             sem_a, sem_b):
        abufs = (abuf0, abuf1)
        bbufs = (bbuf0, bbuf1)
        c = lax.axis_index("c")
        s = lax.axis_index("s")

        pltpu.sync_copy(aidx_hbm.at[c, s], arow)
        pltpu.sync_copy(bidx_hbm.at[c, s], bcol)
        pltpu.sync_copy(w2rot_hbm.at[c], w2v)
        pltpu.sync_copy(featrot_hbm, featv_tab)

        for b in range(2):
            pltpu.async_copy(ab_hbm.at[arow.at[b]], abufs[b], sem_a.at[b])
            pltpu.async_copy(ab_hbm.at[bcol.at[b]], bbufs[b], sem_b.at[b])

        lane = lax.iota(jnp.int32, 16)
        erows = [g * 16 + lane for g in range(CHUNK // 16)]
        zeros8 = tuple(jnp.zeros((16,), jnp.float32) for _ in range(CHUNK // 16))

        def pair_body(kk, _):
            for b in range(2):
                k = kk * 2 + b
                pltpu.make_async_copy(
                    ab_hbm.at[arow.at[k]], abufs[b], sem_a.at[b]
                ).wait()
                pltpu.make_async_copy(
                    ab_hbm.at[bcol.at[k]], bbufs[b], sem_b.at[b]
                ).wait()

                def feat_body(m, accs):
                    featv = featv_tab[pl.ds(m * 16, 16)]
                    w2r = w2v[pl.ds(m * 16, 16)]
                    out = []
                    for g in range(CHUNK // 16):
                        av = plsc.load_gather(abufs[b], [erows[g], featv])
                        bv = plsc.load_gather(bbufs[b], [erows[g], featv])
                        out.append(accs[g] + jnp.maximum(av + bv, 0.0) * w2r)
                    return tuple(out)

                accs = lax.fori_loop(0, DH, feat_body, zeros8, unroll=False)
                for g in range(CHUNK // 16):
                    obuf[k, pl.ds(g * 16, 16)] = accs[g]

                @pl.when(k + 2 < n_chunks)
                def _issue_next():
                    pltpu.async_copy(
                        ab_hbm.at[arow.at[k + 2]], abufs[b], sem_a.at[b]
                    )
                    pltpu.async_copy(
                        ab_hbm.at[bcol.at[k + 2]], bbufs[b], sem_b.at[b]
                    )
            return 0

        lax.fori_loop(0, n_chunks // 2, pair_body, 0, unroll=False)
        pltpu.sync_copy(obuf, out_hbm.at[c, s])

    return elog


def _tc_prep(X0, W4, b4):
    def body(x_ref, w_ref, b_ref, out_ref):
        out_ref[0, 0] = (
            jnp.dot(x_ref[...], w_ref[0, 0], preferred_element_type=jnp.float32)
            + b_ref[0, 0, 0]
        )

    return pl.pallas_call(
        body,
        grid=(4, NC, N // BLKR),
        in_specs=[
            pl.BlockSpec((BLKR, D), lambda m, ci, r: (r, 0)),
            pl.BlockSpec((1, 1, D, DH), lambda m, ci, r: (m, ci, 0, 0)),
            pl.BlockSpec((1, 1, 1, DH), lambda m, ci, r: (m, ci, 0, 0)),
        ],
        out_specs=pl.BlockSpec((1, 1, BLKR, DH), lambda m, ci, r: (m, ci, r, 0)),
        out_shape=jax.ShapeDtypeStruct((4, NC, N, DH), jnp.float32),
    )(X0, W4, b4)


def _tc_smask(E_split, w1, b1, w2, b2, eps):
    def body(e_ref, w1_ref, b1_ref, w2_ref, b2_ref, eps_ref, sm_ref, ps_ref):
        E = jnp.concatenate([e_ref[0], e_ref[1]], axis=-1)
        h = jnp.maximum(
            jnp.dot(E, w1_ref[...], preferred_element_type=jnp.float32)
            + b1_ref[...],
            0.0,
        )
        sm = jnp.dot(h, w2_ref[...], preferred_element_type=jnp.float32) + b2_ref[0]
        e = eps_ref[...]
        gate = (jnp.log(e) - jnp.log(1.0 - e) + sm) / TMP
        m = jax.nn.sigmoid(gate)
        sm_ref[...] = m
        ps_ref[pl.program_id(0), 0] = jnp.sum(m)

    return pl.pallas_call(
        body,
        grid=(N // BLKR,),
        in_specs=[
            pl.BlockSpec((NC, BLKR, DH), lambda r: (0, r, 0)),
            pl.BlockSpec((D, D), lambda r: (0, 0)),
            pl.BlockSpec((1, D), lambda r: (0, 0)),
            pl.BlockSpec((D, 1), lambda r: (0, 0)),
            pl.BlockSpec(memory_space=pltpu.SMEM),
            pl.BlockSpec((BLKR, 1), lambda r: (r, 0)),
        ],
        out_specs=[
            pl.BlockSpec((BLKR, 1), lambda r: (r, 0)),
            pl.BlockSpec(memory_space=pltpu.SMEM),
        ],
        out_shape=[
            jax.ShapeDtypeStruct((N, 1), jnp.float32),
            jax.ShapeDtypeStruct((N // BLKR, 1), jnp.float32),
        ],
    )(E_split, w1, b1, w2, b2, eps)


def _tc_tgate(parts, eps2, vals2, b2):
    M = eps2.shape[0]

    def body(p_ref, eps_ref, v_ref, b2_ref, tm_ref, wv_ref):
        tm = p_ref[0] + p_ref[1] + b2_ref[0]
        e = eps_ref[...]
        gate = (jnp.log(e) - jnp.log(1.0 - e) + tm) / TMP
        m = jax.nn.sigmoid(gate)
        tm_ref[...] = m
        wv_ref[...] = v_ref[...] * m

    return pl.pallas_call(
        body,
        in_specs=[
            pl.BlockSpec((NC, M, CHUNK), lambda: (0, 0, 0)),
            pl.BlockSpec((M, CHUNK), lambda: (0, 0)),
            pl.BlockSpec((M, CHUNK), lambda: (0, 0)),
            pl.BlockSpec(memory_space=pltpu.SMEM),
        ],
        out_specs=[
            pl.BlockSpec((M, CHUNK), lambda: (0, 0)),
            pl.BlockSpec((M, CHUNK), lambda: (0, 0)),
        ],
        out_shape=[
            jax.ShapeDtypeStruct((M, CHUNK), jnp.float32),
            jax.ShapeDtypeStruct((M, CHUNK), jnp.float32),
        ],
    )(parts, eps2, vals2, b2)


def _tc_combine(S_split, mp_split, sm):
    def body(s_ref, mp_ref, sm_ref, out_ref):
        smv = sm_ref[...]
        out_ref[0] = smv * s_ref[0] + (1.0 - smv) * mp_ref[0]

    return pl.pallas_call(
        body,
        grid=(NC, N // BLKR),
        in_specs=[
            pl.BlockSpec((1, BLKR, DH), lambda ci, r: (ci, r, 0)),
            pl.BlockSpec((1, BLKR, DH), lambda ci, r: (ci, r, 0)),
            pl.BlockSpec((BLKR, 1), lambda ci, r: (r, 0)),
        ],
        out_specs=pl.BlockSpec((1, BLKR, DH), lambda ci, r: (ci, r, 0)),
        out_shape=jax.ShapeDtypeStruct((NC, N, DH), jnp.float32),
    )(S_split, mp_split, sm)


def _tc_final(X0, T1, T2, S1, S2):
    def body(x_ref, t1_ref, t2_ref, s1_ref, s2_ref, ot_ref, os_ref):
        def merge(r):
            return jnp.concatenate([r[0], r[1]], axis=-1)

        x = x_ref[...]
        ot_ref[...] = (x + merge(t1_ref) + merge(t2_ref)) * (1.0 / 3.0)
        os_ref[...] = (x + merge(s1_ref) + merge(s2_ref)) * (1.0 / 3.0)

    split_spec = pl.BlockSpec((NC, BLKR, DH), lambda r: (0, r, 0))
    nat_spec = pl.BlockSpec((BLKR, D), lambda r: (r, 0))
    return pl.pallas_call(
        body,
        grid=(N // BLKR,),
        in_specs=[nat_spec, split_spec, split_spec, split_spec, split_spec],
        out_specs=[nat_spec, nat_spec],
        out_shape=[
            jax.ShapeDtypeStruct((N, D), jnp.float32),
            jax.ShapeDtypeStruct((N, D), jnp.float32),
        ],
    )(X0, T1, T2, S1, S2)


def _n_chunks(nnz):
    nch = -(-nnz // (NS * CHUNK))
    return nch + (nch % 2)


def _pad_edges(a, n_chunks):
    npad = NS * n_chunks * CHUNK - a.shape[0]
    return jnp.pad(a, (0, npad)).reshape(NS, n_chunks, CHUNK)


def kernel(crime_embedding, row, col, vals, rw_row, rw_col, rw_vals,
           s_W1, s_b1, s_W2, s_b2, t_W1, t_b1, t_W2, t_b2, t_eps, s_eps):
    X0 = crime_embedding
    nnz = row.shape[0]
    nch = _n_chunks(nnz)
    nch_rw = _n_chunks(rw_row.shape[0])
    spmm_main = _make_spmm(nch)
    spmm_rw = _make_spmm(nch_rw)

    rowp = _pad_edges(row, nch)
    colp = _pad_edges(col, nch)
    packp = _pad_edges((row << 16) | col, nch)
    rw_packp = _pad_edges((rw_row << 16) | rw_col, nch_rw)
    rw_valsp = _pad_edges(rw_vals, nch_rw)

    def spmm(x_split, w):
        o = spmm_main(x_split.reshape(2 * N, DH), packp, _pad_edges(w, nch))
        return o

    X0s = X0.reshape(N, 2, DH).transpose(1, 0, 2)

    W4 = jnp.stack([
        jnp.stack([t_W1[i][half * D:(half + 1) * D, ci * DH:(ci + 1) * DH]
                   for ci in range(NC)])
        for i in range(L) for half in range(2)
    ])
    b4 = jnp.stack([
        jnp.stack([jnp.where(half == 1, t_b1[i][ci * DH:(ci + 1) * DH], 0.0)
                   for ci in range(NC)])
        for i in range(L) for half in range(2)
    ])
    AB = _tc_prep(X0, W4, b4.reshape(4, NC, 1, DH))

    M = nnz // CHUNK
    m_idx = jnp.arange(DH)
    lane_idx = jnp.arange(16)
    featrot = ((m_idx[:, None] // 16) * 16
               + (lane_idx[None, :] + m_idx[:, None] % 16) % 16)
    featrot_flat = featrot.reshape(DH * 16).astype(jnp.int32)
    t_masks, wvals = [], []
    for i in range(L):
        aidx = rowp[None] + jnp.array([4 * i * N, (4 * i + 1) * N],
                                      jnp.int32).reshape(NC, 1, 1, 1)
        bidx = colp[None] + jnp.array([(4 * i + 2) * N, (4 * i + 3) * N],
                                      jnp.int32).reshape(NC, 1, 1, 1)
        w2rot = jnp.stack([
            t_W2[i][ci * DH:(ci + 1) * DH, 0][featrot_flat]
            for ci in range(NC)
        ])
        parts = _make_edge_logit(nch)(
            AB.reshape(4 * NC * N, DH), aidx, bidx, w2rot, featrot_flat
        )
        parts = parts.reshape(NC, NS * nch * CHUNK)[:, :nnz].reshape(NC, M, CHUNK)
        tm, wv = _tc_tgate(
            parts,
            t_eps[i].reshape(M, CHUNK),
            vals.reshape(M, CHUNK),
            t_b2[i].reshape(1),
        )
        t_masks.append(tm.reshape(nnz))
        wvals.append(wv.reshape(nnz))

    E1 = spmm(X0s, vals)
    E2 = spmm(E1, vals)
    s_masks, s_partials = [], []
    for i, E in enumerate((E1, E2)):
        sm, ps = _tc_smask(
            E, s_W1[i], s_b1[i].reshape(1, D), s_W2[i], s_b2[i], s_eps[i]
        )
        s_masks.append(sm)
        s_partials.append(ps)

    T1 = spmm(X0s, wvals[0])
    T2 = spmm(T1, wvals[1])

    S = X0s
    outs = []
    for i in range(L):
        mp = spmm_rw(S.reshape(2 * N, DH), rw_packp, rw_valsp)
        C = _tc_combine(S, mp, s_masks[i])
        S = spmm(C, vals)
        outs.append(S)

    out_t, out_s = _tc_final(X0, T1, T2, outs[0], outs[1])

    s_reg = (s_partials[0].sum() + s_partials[1].sum()) / N / L
    t_reg = jnp.zeros((), jnp.float32)
    return (out_t, out_s, t_reg, s_reg, t_masks[-1])
